# Initial kernel scaffold; baseline (speedup 1.0000x reference)
import jax
import jax.numpy as jnp
from jax.experimental import pallas as pl


def kernel(relation_logits, frame_type_ids, W, b):
    raise NotImplementedError("write your pallas kernel here")


# speedup vs baseline: 6.9179x; 6.9179x over previous
import functools

import jax
import jax.numpy as jnp
from jax import lax
from jax.experimental import pallas as pl
from jax.experimental.pallas import tpu as pltpu
from jax.experimental.pallas import tpu_sc as plsc

B = 16384
RD = 14
CD = 64
F = 4
RDA = RD + 1

TC_BLK = 1024
TC_GRID = B // TC_BLK


def _tc_dense(x_ref, w_ref, y_ref):
    y_ref[...] = jnp.dot(x_ref[...], w_ref[...],
                         preferred_element_type=jnp.float32)


NC = 2
NS = 16
L = 16
NW = NC * NS
CHUNK = B // NW
SEG = 128
NSEG = CHUNK // SEG


def _sc_route(yall_hbm, ids_hbm, out_hbm, ids_v, idx_v, rows_v, sem):
    wid = lax.axis_index("s") * NC + lax.axis_index("c")
    base = wid * CHUNK
    pltpu.sync_copy(ids_hbm.at[pl.ds(base, CHUNK)], ids_v)
    lane = lax.iota(jnp.int32, L)
    for s in range(NSEG):
        for j in range(SEG // L):
            off = s * SEG + j * L
            ids16 = ids_v[pl.ds(off, L)]
            idx_v[s, pl.ds(j * L, L)] = (base + off) * F + lane * F + ids16
    copies = [
        pltpu.async_copy(yall_hbm.at[idx_v.at[s]],
                         rows_v.at[pl.ds(s * SEG, SEG)], sem)
        for s in range(NSEG)
    ]
    for c in copies:
        c.wait()
    pltpu.sync_copy(rows_v, out_hbm.at[pl.ds(base, CHUNK)])


def kernel(relation_logits, frame_type_ids, W, b):
    xaug = jnp.concatenate(
        [relation_logits, jnp.ones((B, 1), jnp.float32)], axis=1)
    wt = jnp.concatenate([W.transpose(0, 2, 1), b[:, None, :]], axis=1)
    wall = wt.transpose(1, 0, 2).reshape(RDA, F * CD)

    yall = pl.pallas_call(
        _tc_dense,
        grid=(TC_GRID,),
        in_specs=[
            pl.BlockSpec((TC_BLK, RDA), lambda i: (i, 0)),
            pl.BlockSpec((RDA, F * CD), lambda i: (0, 0)),
        ],
        out_specs=pl.BlockSpec((TC_BLK, F * CD), lambda i: (i, 0)),
        out_shape=jax.ShapeDtypeStruct((B, F * CD), jnp.float32),
    )(xaug, wall)

    yall_rows = yall.reshape(B * F, CD)

    sc_call = functools.partial(
        pl.kernel,
        mesh=plsc.VectorSubcoreMesh(core_axis_name="c", subcore_axis_name="s"),
        compiler_params=pltpu.CompilerParams(use_tc_tiling_on_sc=False),
        out_type=jax.ShapeDtypeStruct((B, CD), jnp.float32),
        scratch_types=[
            pltpu.VMEM((CHUNK,), jnp.int32),
            pltpu.VMEM((NSEG, SEG), jnp.int32),
            pltpu.VMEM((CHUNK, CD), jnp.float32),
            pltpu.SemaphoreType.DMA,
        ],
    )(_sc_route)
    return sc_call(yall_rows, frame_type_ids)


# speedup vs baseline: 7.6923x; 1.1119x over previous
import functools

import jax
import jax.numpy as jnp
from jax import lax
from jax.experimental import pallas as pl
from jax.experimental.pallas import tpu as pltpu
from jax.experimental.pallas import tpu_sc as plsc

B = 16384
RD = 14
CD = 64
F = 4
RDA = RD + 1

TC_BLK = 1024
TC_GRID = B // TC_BLK


def _tc_dense(x_ref, w_ref, y_ref):
    x = x_ref[...]
    y_ref[0] = jnp.dot(x, w_ref[:, :128], preferred_element_type=jnp.float32)
    y_ref[1] = jnp.dot(x, w_ref[:, 128:], preferred_element_type=jnp.float32)


NC = 2
NS = 16
L = 16
NW = NC * NS
CHUNK = B // NW
SEG = 128
NSEG = CHUNK // SEG


def _sc_route(yall_hbm, ids_hbm, out_hbm, ids_v, idx_v, rows_v, out_v, sem):
    wid = lax.axis_index("s") * NC + lax.axis_index("c")
    base = wid * CHUNK
    pltpu.sync_copy(ids_hbm.at[pl.ds(base, CHUNK)], ids_v)
    lane = lax.iota(jnp.int32, L)
    for s in range(NSEG):
        for j in range(SEG // L):
            off = s * SEG + j * L
            ids16 = ids_v[pl.ds(off, L)]
            idx_v[s, pl.ds(j * L, L)] = (
                (ids16 >> 1) * B + (base + off) + lane)
    copies = [
        pltpu.async_copy(yall_hbm.at[idx_v.at[s]],
                         rows_v.at[pl.ds(s * SEG, SEG)], sem)
        for s in range(NSEG)
    ]
    for c in copies:
        c.wait()

    def body(g, _):
        ids16 = ids_v[pl.ds(g * L, L)]
        for lt in range(L):
            t = g * L + lt
            hoff = (ids16[lt] & 1) * CD
            for k in range(CD // L):
                out_v[t, pl.ds(k * L, L)] = rows_v[t, pl.ds(hoff + k * L, L)]
        return 0

    lax.fori_loop(0, CHUNK // L, body, 0)
    pltpu.sync_copy(out_v, out_hbm.at[pl.ds(base, CHUNK)])


def kernel(relation_logits, frame_type_ids, W, b):
    xaug = jnp.concatenate(
        [relation_logits, jnp.ones((B, 1), jnp.float32)], axis=1)
    wt = jnp.concatenate([W.transpose(0, 2, 1), b[:, None, :]], axis=1)
    wall = wt.transpose(1, 0, 2).reshape(RDA, F * CD)

    yall = pl.pallas_call(
        _tc_dense,
        grid=(TC_GRID,),
        in_specs=[
            pl.BlockSpec((TC_BLK, RDA), lambda i: (i, 0)),
            pl.BlockSpec((RDA, F * CD), lambda i: (0, 0)),
        ],
        out_specs=pl.BlockSpec((2, TC_BLK, 2 * CD), lambda i: (0, i, 0)),
        out_shape=jax.ShapeDtypeStruct((2, B, 2 * CD), jnp.float32),
    )(xaug, wall)

    yall_rows = yall.reshape(2 * B, 2 * CD)

    sc_call = functools.partial(
        pl.kernel,
        mesh=plsc.VectorSubcoreMesh(core_axis_name="c", subcore_axis_name="s"),
        compiler_params=pltpu.CompilerParams(use_tc_tiling_on_sc=False),
        out_type=jax.ShapeDtypeStruct((B, CD), jnp.float32),
        scratch_types=[
            pltpu.VMEM((CHUNK,), jnp.int32),
            pltpu.VMEM((NSEG, SEG), jnp.int32),
            pltpu.VMEM((CHUNK, 2 * CD), jnp.float32),
            pltpu.VMEM((CHUNK, CD), jnp.float32),
            pltpu.SemaphoreType.DMA,
        ],
    )(_sc_route)
    return sc_call(yall_rows, frame_type_ids)


# speedup vs baseline: 9.4190x; 1.2245x over previous
import functools

import jax
import jax.numpy as jnp
from jax import lax
from jax.experimental import pallas as pl
from jax.experimental.pallas import tpu as pltpu
from jax.experimental.pallas import tpu_sc as plsc

B = 16384
RD = 14
CD = 64
F = 4

TC_BLK = 1024
TC_GRID = B // TC_BLK


def _tc_dense(x_ref, w_ref, b_ref, y_ref):
    x = x_ref[...]
    bias = b_ref[...]
    y_ref[0] = (jnp.dot(x, w_ref[:, :128], preferred_element_type=jnp.float32)
                + bias[:, :128])
    y_ref[1] = (jnp.dot(x, w_ref[:, 128:], preferred_element_type=jnp.float32)
                + bias[:, 128:])


NC = 2
NS = 16
L = 16
NW = NC * NS
CHUNK = B // NW
SEG = 128
NSEG = CHUNK // SEG


def _sc_route(yall_hbm, ids_hbm, out_hbm, ids_v, idx_v, rows_v, sem):
    wid = lax.axis_index("s") * NC + lax.axis_index("c")
    base = wid * CHUNK
    pltpu.sync_copy(ids_hbm.at[pl.ds(base, CHUNK)], ids_v)
    lane = lax.iota(jnp.int32, L)
    for s in range(NSEG):
        for j in range(SEG // L):
            off = s * SEG + j * L
            ids16 = ids_v[pl.ds(off, L)]
            idx_v[s, pl.ds(j * L, L)] = (
                (ids16 >> 1) * (2 * B) + 2 * (base + off) + 2 * lane
                + (ids16 & 1))
    copies = [
        pltpu.async_copy(yall_hbm.at[idx_v.at[s]],
                         rows_v.at[pl.ds(s * SEG, SEG)], sem)
        for s in range(NSEG)
    ]
    for c in copies:
        c.wait()
    pltpu.sync_copy(rows_v, out_hbm.at[pl.ds(base, CHUNK)])


def kernel(relation_logits, frame_type_ids, W, b):
    wall = W.transpose(0, 2, 1).transpose(1, 0, 2).reshape(RD, F * CD)
    ball = b.reshape(1, F * CD)

    yall = pl.pallas_call(
        _tc_dense,
        grid=(TC_GRID,),
        in_specs=[
            pl.BlockSpec((TC_BLK, RD), lambda i: (i, 0)),
            pl.BlockSpec((RD, F * CD), lambda i: (0, 0)),
            pl.BlockSpec((1, F * CD), lambda i: (0, 0)),
        ],
        out_specs=pl.BlockSpec((2, TC_BLK, 2 * CD), lambda i: (0, i, 0)),
        out_shape=jax.ShapeDtypeStruct((2, B, 2 * CD), jnp.float32),
    )(relation_logits, wall, ball)

    yall_rows = yall.reshape(F * B, CD)

    sc_call = functools.partial(
        pl.kernel,
        mesh=plsc.VectorSubcoreMesh(core_axis_name="c", subcore_axis_name="s"),
        compiler_params=pltpu.CompilerParams(use_tc_tiling_on_sc=False),
        out_type=jax.ShapeDtypeStruct((B, CD), jnp.float32),
        scratch_types=[
            pltpu.VMEM((CHUNK,), jnp.int32),
            pltpu.VMEM((NSEG, SEG), jnp.int32),
            pltpu.VMEM((CHUNK, CD), jnp.float32),
            pltpu.SemaphoreType.DMA,
        ],
    )(_sc_route)
    return sc_call(yall_rows, frame_type_ids)
